# Initial kernel scaffold; baseline (speedup 1.0000x reference)
import jax
import jax.numpy as jnp
from jax.experimental import pallas as pl


def kernel(hidden_states):
    raise NotImplementedError("write your pallas kernel here")


# speedup vs baseline: 24.6927x; 24.6927x over previous
import jax
import jax.numpy as jnp
from jax.experimental import pallas as pl

N_TOK = 8192
D = 64
N_EXP = 64
BLK = 1024


def _router_block(x_ref, sel_ref, w_ref, mask_ref):
    x = x_ref[:, :D]
    i32 = jnp.int32
    sign_code = jnp.where(x > 0.0, i32(4), jnp.where(x < 0.0, i32(12), i32(0)))
    mag = jnp.minimum(jnp.abs(x), jnp.float32(7.0)).astype(i32)
    dv = sign_code | mag
    w = jax.lax.broadcasted_iota(i32, (BLK, D), 1) + i32(1)
    t = dv * w
    t = t[:, :32] ^ t[:, 32:]
    t = t[:, :16] ^ t[:, 16:]
    t = t[:, :8] ^ t[:, 8:]
    t = t[:, :4] ^ t[:, 4:]
    t = t[:, :2] ^ t[:, 2:]
    key = t[:, :1] ^ t[:, 1:]
    e0 = key & i32(63)
    e1 = e0 ^ i32(1)
    sel_ref[...] = jnp.concatenate([e0, e1], axis=1)
    w_ref[...] = jnp.full((BLK, 2), 0.5, dtype=jnp.float32)
    cols = jax.lax.broadcasted_iota(i32, (BLK, N_EXP), 1)
    mask_ref[...] = jnp.where(
        (cols == e0) | (cols == e1), jnp.float32(0.5), jnp.float32(0.0)
    )


def kernel(hidden_states):
    grid = (N_TOK // BLK,)
    sel32, weights, masks = pl.pallas_call(
        _router_block,
        grid=grid,
        in_specs=[pl.BlockSpec((BLK, 128), lambda i: (i, i * 0))],
        out_specs=[
            pl.BlockSpec((BLK, 2), lambda i: (i, i * 0)),
            pl.BlockSpec((BLK, 2), lambda i: (i, i * 0)),
            pl.BlockSpec((BLK, N_EXP), lambda i: (i, i * 0)),
        ],
        out_shape=[
            jax.ShapeDtypeStruct((N_TOK, 2), jnp.int32),
            jax.ShapeDtypeStruct((N_TOK, 2), jnp.float32),
            jax.ShapeDtypeStruct((N_TOK, N_EXP), jnp.float32),
        ],
    )(hidden_states)
    return sel32.astype(jnp.int64), weights, masks


# speedup vs baseline: 26.5681x; 1.0759x over previous
import functools

import jax
import jax.numpy as jnp
from jax import lax
from jax.experimental import pallas as pl
from jax.experimental.pallas import tpu as pltpu
from jax.experimental.pallas import tpu_sc as plsc

N_TOK = 8192
HID = 1024
D = 64
N_EXP = 64
NC = 2
NS = 16
NW = NC * NS
TPW = N_TOK // NW
PSTR = 257

_i32 = jnp.int32
_f32 = jnp.float32


def _sc_router(hs_hbm, sel_hbm, w_hbm, mask_hbm, x_v, p_v, sel_v, w_v, mask_v):
    wid = lax.axis_index("s") * _i32(NC) + lax.axis_index("c")
    base = wid * _i32(TPW)

    pltpu.sync_copy(hs_hbm.at[pl.ds(base, TPW), pl.ds(0, 128)], x_v)

    iota = lax.iota(_i32, 16)
    p_idx0 = iota * PSTR
    wvec = [iota + _i32(16 * j + 1) for j in range(4)]
    zero16 = jnp.zeros((16,), _f32)

    def phase1(tb, carry):
        for u in range(8):
            t = tb * _i32(8) + _i32(u)
            acc = jnp.zeros((16,), _i32)
            trow = t * _i32(D)
            for j in range(4):
                v = x_v[t, pl.ds(16 * j, 16)]
                s = jnp.where(v > _f32(0.0), _i32(4),
                              jnp.where(v < _f32(0.0), _i32(12), _i32(0)))
                m = jnp.minimum(jnp.abs(v), _f32(7.0)).astype(_i32)
                acc = acc ^ ((s | m) * wvec[j])
                mask_v[pl.ds(trow + _i32(16 * j), 16)] = zero16
            plsc.store_scatter(p_v, [p_idx0 + t], acc)
        return carry

    lax.fori_loop(_i32(0), _i32(TPW // 8), phase1, _i32(0))

    half16 = jnp.full((16,), 0.5, _f32)
    sel_idx0 = iota * _i32(2)
    mrow_idx0 = iota * _i32(D)

    def phase2(g, carry):
        t0 = g * _i32(16)
        key = p_v[pl.ds(t0, 16)]
        for l in range(1, 16):
            key = key ^ p_v[pl.ds(t0 + _i32(l * PSTR), 16)]
        e0 = key & _i32(63)
        e1 = e0 ^ _i32(1)
        t0x2 = t0 * _i32(2)
        plsc.store_scatter(sel_v, [sel_idx0 + t0x2], e0)
        plsc.store_scatter(sel_v, [sel_idx0 + t0x2 + _i32(1)], e1)
        midx = mrow_idx0 + t0 * _i32(D) + e0
        plsc.store_scatter(mask_v, [midx], half16)
        plsc.store_scatter(mask_v, [midx ^ _i32(1)], half16)
        w_v[pl.ds(t0x2, 16)] = half16
        w_v[pl.ds(t0x2 + _i32(16), 16)] = half16
        return carry

    lax.fori_loop(_i32(0), _i32(TPW // 16), phase2, _i32(0))

    pltpu.sync_copy(sel_v, sel_hbm.at[pl.ds(base * _i32(2), TPW * 2)])
    pltpu.sync_copy(w_v, w_hbm.at[pl.ds(base * _i32(2), TPW * 2)])
    pltpu.sync_copy(mask_v, mask_hbm.at[pl.ds(base * _i32(D), TPW * D)])


@functools.partial(
    pl.kernel,
    out_type=[
        jax.ShapeDtypeStruct((N_TOK * 2,), _i32),
        jax.ShapeDtypeStruct((N_TOK * 2,), _f32),
        jax.ShapeDtypeStruct((N_TOK * N_EXP,), _f32),
    ],
    mesh=plsc.VectorSubcoreMesh(core_axis_name="c", subcore_axis_name="s"),
    compiler_params=pltpu.CompilerParams(needs_layout_passes=False),
    scratch_types=[
        pltpu.VMEM((TPW, 128), _f32),
        pltpu.VMEM((16 * PSTR,), _i32),
        pltpu.VMEM((TPW * 2,), _i32),
        pltpu.VMEM((TPW * 2,), _f32),
        pltpu.VMEM((TPW * D,), _f32),
    ],
)
def _sc_call(hs_hbm, sel_hbm, w_hbm, mask_hbm, x_v, p_v, sel_v, w_v, mask_v):
    _sc_router(hs_hbm, sel_hbm, w_hbm, mask_hbm, x_v, p_v, sel_v, w_v, mask_v)


def kernel(hidden_states):
    sel_flat, w_flat, mask_flat = _sc_call(hidden_states)
    return (
        sel_flat.reshape(N_TOK, 2).astype(jnp.int64),
        w_flat.reshape(N_TOK, 2),
        mask_flat.reshape(N_TOK, N_EXP),
    )


# speedup vs baseline: 65.6673x; 2.4717x over previous
import functools

import jax
import jax.numpy as jnp
from jax import lax
from jax.experimental import pallas as pl
from jax.experimental.pallas import tpu as pltpu
from jax.experimental.pallas import tpu_sc as plsc

N_TOK = 8192
D = 64
N_EXP = 64
NC = 2
NS = 16
NW = NC * NS
TPW = N_TOK // NW
PSTR = 257

_i32 = jnp.int32
_f32 = jnp.float32

_F8_HALF = 0x30
_BF16_HALF_PAIR = 0x3F003F00


def _sc_router(hs_hbm, sel_hbm, w_hbm, mask_hbm, x_v, p_v, sel_v, w_v, mask_v):
    wid = lax.axis_index("s") * _i32(NC) + lax.axis_index("c")
    base = wid * _i32(TPW)

    pltpu.sync_copy(hs_hbm.at[pl.ds(base, TPW), pl.ds(0, 128)], x_v)

    iota = lax.iota(_i32, 16)
    p_idx0 = iota * _i32(PSTR)
    wvec = [iota + _i32(16 * j + 1) for j in range(4)]
    zero16 = jnp.zeros((16,), _i32)

    def phase1(tb, carry):
        for u in range(8):
            t = tb * _i32(8) + _i32(u)
            acc = jnp.zeros((16,), _i32)
            for j in range(4):
                v = x_v[t, pl.ds(16 * j, 16)]
                s = jnp.where(v > _f32(0.0), _i32(4),
                              jnp.where(v < _f32(0.0), _i32(12), _i32(0)))
                m = jnp.minimum(jnp.abs(v), _f32(7.0)).astype(_i32)
                acc = acc ^ ((s | m) * wvec[j])
            mask_v[pl.ds(t * _i32(16), 16)] = zero16
            plsc.store_scatter(p_v, [p_idx0 + t], acc)
        return carry

    lax.fori_loop(_i32(0), _i32(TPW // 8), phase1, _i32(0))

    wpair16 = jnp.full((16,), _BF16_HALF_PAIR, _i32)
    lo_pair = jnp.full((16,), _F8_HALF | (_F8_HALF << 8), _i32)
    hi_pair = jnp.full((16,), (_F8_HALF << 16) | (_F8_HALF << 24), _i32)
    iota16 = iota * _i32(16)

    def phase2(g, carry):
        t0 = g * _i32(16)
        key = p_v[pl.ds(t0, 16)]
        for l in range(1, 16):
            key = key ^ p_v[pl.ds(t0 + _i32(l * PSTR), 16)]
        e0 = key & _i32(63)
        e1 = e0 ^ _i32(1)
        sel_v[pl.ds(t0, 16)] = e0 | (e1 << _i32(16))
        w_v[pl.ds(t0, 16)] = wpair16
        val = jnp.where((e0 & _i32(2)) == _i32(0), lo_pair, hi_pair)
        midx = t0 * _i32(16) + iota16 + (e0 >> _i32(2))
        plsc.store_scatter(mask_v, [midx], val)
        return carry

    lax.fori_loop(_i32(0), _i32(TPW // 16), phase2, _i32(0))

    pltpu.sync_copy(sel_v, sel_hbm.at[pl.ds(base, TPW)])
    pltpu.sync_copy(w_v, w_hbm.at[pl.ds(base, TPW)])
    pltpu.sync_copy(mask_v, mask_hbm.at[pl.ds(base * _i32(16), TPW * 16)])


@functools.partial(
    pl.kernel,
    out_type=[
        jax.ShapeDtypeStruct((N_TOK,), _i32),
        jax.ShapeDtypeStruct((N_TOK,), _i32),
        jax.ShapeDtypeStruct((N_TOK * 16,), _i32),
    ],
    mesh=plsc.VectorSubcoreMesh(core_axis_name="c", subcore_axis_name="s"),
    compiler_params=pltpu.CompilerParams(needs_layout_passes=False),
    scratch_types=[
        pltpu.VMEM((TPW, 128), _f32),
        pltpu.VMEM((16 * PSTR,), _i32),
        pltpu.VMEM((TPW,), _i32),
        pltpu.VMEM((TPW,), _i32),
        pltpu.VMEM((TPW * 16,), _i32),
    ],
)
def _sc_call(hs_hbm, sel_hbm, w_hbm, mask_hbm, x_v, p_v, sel_v, w_v, mask_v):
    _sc_router(hs_hbm, sel_hbm, w_hbm, mask_hbm, x_v, p_v, sel_v, w_v, mask_v)


def kernel(hidden_states):
    sel_p, w_p, mask_p = _sc_call(hidden_states)
    sel = lax.bitcast_convert_type(sel_p, jnp.int16).astype(jnp.int64)
    w = lax.bitcast_convert_type(w_p, jnp.bfloat16).astype(jnp.float32)
    mask = lax.bitcast_convert_type(
        mask_p.reshape(N_TOK, 16), jnp.float8_e4m3fn
    ).reshape(N_TOK, N_EXP).astype(jnp.float32)
    return sel, w, mask


# speedup vs baseline: 65.6770x; 1.0001x over previous
import functools

import jax
import jax.numpy as jnp
from jax import lax
from jax.experimental import pallas as pl
from jax.experimental.pallas import tpu as pltpu
from jax.experimental.pallas import tpu_sc as plsc

N_TOK = 8192
D = 64
N_EXP = 64
NC = 2
NS = 16
NW = NC * NS
TPW = N_TOK // NW
PSTR = 257

_i32 = jnp.int32
_f32 = jnp.float32

_F8_HALF = 0x30
_BF16_HALF_PAIR = 0x3F003F00


def _sc_router(hs_hbm, sel_hbm, w_hbm, mask_hbm, x_v, p_v, sel_v, w_v, mask_v):
    wid = lax.axis_index("s") * _i32(NC) + lax.axis_index("c")
    base = wid * _i32(TPW)

    pltpu.sync_copy(hs_hbm.at[pl.ds(base, TPW), pl.ds(0, 128)], x_v)

    iota = lax.iota(_i32, 16)
    p_idx0 = iota * _i32(PSTR)
    wvec = [iota + _i32(16 * j + 1) for j in range(4)]
    zero16 = jnp.zeros((16,), _i32)

    def phase1(tb, carry):
        for u in range(8):
            t = tb * _i32(8) + _i32(u)
            acc = jnp.zeros((16,), _i32)
            for j in range(4):
                v = x_v[t, pl.ds(16 * j, 16)]
                s = jnp.where(v > _f32(0.0), _i32(4),
                              jnp.where(v < _f32(0.0), _i32(12), _i32(0)))
                m = jnp.minimum(jnp.abs(v), _f32(7.0)).astype(_i32)
                acc = acc ^ ((s | m) * wvec[j])
            mask_v[pl.ds(t * _i32(16), 16)] = zero16
            plsc.store_scatter(p_v, [p_idx0 + t], acc)
        return carry

    lax.fori_loop(_i32(0), _i32(TPW // 8), phase1, _i32(0))

    wpair16 = jnp.full((16,), _BF16_HALF_PAIR, _i32)
    lo_pair = jnp.full((16,), _F8_HALF | (_F8_HALF << 8), _i32)
    hi_pair = jnp.full((16,), (_F8_HALF << 16) | (_F8_HALF << 24), _i32)
    iota16 = iota * _i32(16)

    def phase2(g, carry):
        t0 = g * _i32(16)
        key = p_v[pl.ds(t0, 16)]
        for l in range(1, 16):
            key = key ^ p_v[pl.ds(t0 + _i32(l * PSTR), 16)]
        e0 = key & _i32(63)
        e1 = e0 ^ _i32(1)
        sel_v[pl.ds(t0, 16)] = e0 | (e1 << _i32(16))
        w_v[pl.ds(t0, 16)] = wpair16
        val = jnp.where((e0 & _i32(2)) == _i32(0), lo_pair, hi_pair)
        midx = t0 * _i32(16) + iota16 + (e0 >> _i32(2))
        plsc.store_scatter(mask_v, [midx], val)
        return carry

    lax.fori_loop(_i32(0), _i32(TPW // 16), phase2, _i32(0))

    pltpu.sync_copy(sel_v, sel_hbm.at[pl.ds(base, TPW)])
    pltpu.sync_copy(w_v, w_hbm.at[pl.ds(base, TPW)])
    pltpu.sync_copy(mask_v, mask_hbm.at[pl.ds(base * _i32(16), TPW * 16)])


@functools.partial(
    pl.kernel,
    out_type=[
        jax.ShapeDtypeStruct((N_TOK,), _i32),
        jax.ShapeDtypeStruct((N_TOK,), _i32),
        jax.ShapeDtypeStruct((N_TOK * 16,), _i32),
    ],
    mesh=plsc.VectorSubcoreMesh(core_axis_name="c", subcore_axis_name="s"),
    compiler_params=pltpu.CompilerParams(
        needs_layout_passes=False, skip_device_barrier=True
    ),
    scratch_types=[
        pltpu.VMEM((TPW, 128), _f32),
        pltpu.VMEM((16 * PSTR,), _i32),
        pltpu.VMEM((TPW,), _i32),
        pltpu.VMEM((TPW,), _i32),
        pltpu.VMEM((TPW * 16,), _i32),
    ],
)
def _sc_call(hs_hbm, sel_hbm, w_hbm, mask_hbm, x_v, p_v, sel_v, w_v, mask_v):
    _sc_router(hs_hbm, sel_hbm, w_hbm, mask_hbm, x_v, p_v, sel_v, w_v, mask_v)


def kernel(hidden_states):
    sel_p, w_p, mask_p = _sc_call(hidden_states)
    sel = lax.bitcast_convert_type(sel_p, jnp.int16).astype(jnp.int64)
    w = lax.bitcast_convert_type(w_p, jnp.bfloat16).astype(jnp.float32)
    mask = lax.bitcast_convert_type(
        mask_p.reshape(N_TOK, 16), jnp.float8_e4m3fn
    ).reshape(N_TOK, N_EXP).astype(jnp.float32)
    return sel, w, mask


# speedup vs baseline: 65.7139x; 1.0006x over previous
import functools

import jax
import jax.numpy as jnp
from jax import lax
from jax.experimental import pallas as pl
from jax.experimental.pallas import tpu as pltpu
from jax.experimental.pallas import tpu_sc as plsc

N_TOK = 8192
D = 64
N_EXP = 64
NC = 2
NS = 16
NW = NC * NS
TPW = N_TOK // NW
PSTR = 257

_i32 = jnp.int32
_f32 = jnp.float32

_F8_HALF = 0x30
_BF16_HALF_PAIR = 0x3F003F00


def _sc_router(hs_hbm, sel_hbm, w_hbm, mask_hbm, x_v, p_v, sel_v, w_v, mask_v):
    wid = lax.axis_index("s") * _i32(NC) + lax.axis_index("c")
    base = wid * _i32(TPW)

    pltpu.sync_copy(hs_hbm.at[pl.ds(base, TPW), pl.ds(0, 128)], x_v)

    iota = lax.iota(_i32, 16)
    p_idx0 = iota * _i32(PSTR)
    wvec = [iota + _i32(16 * j + 1) for j in range(4)]
    zero16 = jnp.zeros((16,), _i32)

    def phase1(tb, carry):
        for u in range(8):
            t = tb * _i32(8) + _i32(u)
            acc = jnp.zeros((16,), _i32)
            for j in range(4):
                v = x_v[t, pl.ds(16 * j, 16)]
                s = jnp.where(v > _f32(0.0), _i32(4),
                              jnp.where(v < _f32(0.0), _i32(12), _i32(0)))
                m = jnp.minimum(jnp.abs(v), _f32(7.0)).astype(_i32)
                acc = acc ^ ((s | m) * wvec[j])
            mask_v[pl.ds(t * _i32(16), 16)] = zero16
            plsc.store_scatter(p_v, [p_idx0 + t], acc)
        return carry

    lax.fori_loop(_i32(0), _i32(TPW // 8), phase1, _i32(0))

    wpair16 = jnp.full((16,), _BF16_HALF_PAIR, _i32)
    lo_pair = jnp.full((16,), _F8_HALF | (_F8_HALF << 8), _i32)
    hi_pair = jnp.full((16,), (_F8_HALF << 16) | (_F8_HALF << 24), _i32)
    iota16 = iota * _i32(16)

    def phase2(g, carry):
        t0 = g * _i32(16)
        key = p_v[pl.ds(t0, 16)]
        for l in range(1, 16):
            key = key ^ p_v[pl.ds(t0 + _i32(l * PSTR), 16)]
        e0 = key & _i32(63)
        e1 = e0 ^ _i32(1)
        sel_v[pl.ds(t0, 16)] = e0 | (e1 << _i32(16))
        w_v[pl.ds(t0, 16)] = wpair16
        val = jnp.where((e0 & _i32(2)) == _i32(0), lo_pair, hi_pair)
        midx = t0 * _i32(16) + iota16 + (e0 >> _i32(2))
        plsc.store_scatter(mask_v, [midx], val)
        return carry

    lax.fori_loop(_i32(0), _i32(TPW // 16), phase2, _i32(0))

    pltpu.sync_copy(sel_v, sel_hbm.at[pl.ds(base, TPW)])
    pltpu.sync_copy(w_v, w_hbm.at[pl.ds(base, TPW)])
    pltpu.sync_copy(mask_v, mask_hbm.at[pl.ds(base * _i32(16), TPW * 16)])


@functools.partial(
    pl.kernel,
    out_type=[
        jax.ShapeDtypeStruct((N_TOK,), _i32),
        jax.ShapeDtypeStruct((N_TOK,), _i32),
        jax.ShapeDtypeStruct((N_TOK * 16,), _i32),
    ],
    mesh=plsc.VectorSubcoreMesh(core_axis_name="c", subcore_axis_name="s"),
    compiler_params=pltpu.CompilerParams(needs_layout_passes=False),
    scratch_types=[
        pltpu.VMEM((TPW, 128), _f32),
        pltpu.VMEM((16 * PSTR,), _i32),
        pltpu.VMEM((TPW,), _i32),
        pltpu.VMEM((TPW,), _i32),
        pltpu.VMEM((TPW * 16,), _i32),
    ],
)
def _sc_call(hs_hbm, sel_hbm, w_hbm, mask_hbm, x_v, p_v, sel_v, w_v, mask_v):
    _sc_router(hs_hbm, sel_hbm, w_hbm, mask_hbm, x_v, p_v, sel_v, w_v, mask_v)


def kernel(hidden_states):
    sel_p, w_p, mask_p = _sc_call(hidden_states)
    sel = lax.bitcast_convert_type(sel_p, jnp.int16).astype(jnp.int64)
    w = lax.bitcast_convert_type(w_p, jnp.bfloat16).astype(jnp.float32)
    mask = lax.bitcast_convert_type(
        mask_p.reshape(N_TOK, 16), jnp.float8_e4m3fn
    ).reshape(N_TOK, N_EXP).astype(jnp.float32)
    return sel, w, mask
